# baseline (speedup 1.0000x reference)
import functools

import jax
import jax.numpy as jnp
from jax import lax
from jax.experimental import pallas as pl
from jax.experimental.pallas import tpu as pltpu
from jax.experimental.pallas import tpu_sc as plsc

VOCAB = 1000000
DIM = 64
BATCH = 16384

_info = plsc.get_sparse_core_info()
NC = _info.num_cores
NS = _info.num_subcores
NW = NC * NS
B_PER_W = BATCH // NW
CHUNK = 128
NCHUNK = B_PER_W // CHUNK
L = 16

_mesh = plsc.VectorSubcoreMesh(core_axis_name="c", subcore_axis_name="s")


@functools.partial(
    pl.kernel,
    mesh=_mesh,
    out_type=jax.ShapeDtypeStruct((BATCH, 2 * DIM), jnp.float32),
    compiler_params=pltpu.CompilerParams(use_tc_tiling_on_sc=True),
    scratch_types=[
        pltpu.VMEM((NCHUNK, CHUNK), jnp.int32),
        pltpu.VMEM((NCHUNK, CHUNK), jnp.int32),
        pltpu.VMEM((B_PER_W, 2 * DIM), jnp.float32),
        pltpu.SemaphoreType.DMA,
    ],
)
def _gather(table2, idx_hbm, out_hbm, idx_v, bid_v, rows_v, sem):
    wid = lax.axis_index("s") * NC + lax.axis_index("c")
    base = wid * B_PER_W
    pltpu.sync_copy(idx_hbm.at[wid], idx_v)
    for t in range(NCHUNK):
        for u in range(CHUNK // L):
            bid_v[t, pl.ds(u * L, L)] = idx_v[t, pl.ds(u * L, L)] >> 1
    copies = [
        pltpu.async_copy(
            table2.at[bid_v.at[t]],
            rows_v.at[pl.ds(t * CHUNK, CHUNK), :],
            sem,
        )
        for t in range(NCHUNK)
    ]
    for c in copies:
        c.wait()
    pltpu.sync_copy(rows_v, out_hbm.at[pl.ds(base, B_PER_W)])


def kernel(x, table):
    idx = x.reshape(BATCH).astype(jnp.int32)
    table2 = table.reshape(VOCAB // 2, 2 * DIM)
    pairs = _gather(table2, idx.reshape(NW, NCHUNK, CHUNK))
    odd = (idx & 1)[:, None].astype(jnp.bool_)
    return jnp.where(odd, pairs[:, DIM:], pairs[:, :DIM])


# speedup vs baseline: 2.6687x; 2.6687x over previous
import functools

import jax
import jax.numpy as jnp
from jax import lax
from jax.experimental import pallas as pl
from jax.experimental.pallas import tpu as pltpu
from jax.experimental.pallas import tpu_sc as plsc

VOCAB = 1000000
DIM = 64
BATCH = 16384
W = 80

_info = plsc.get_sparse_core_info()
NC = _info.num_cores
NS = _info.num_subcores
NW = NC * NS
L = 16
STRIPE = VOCAB // NW
NBUF = BATCH + 32
TRASH = BATCH
REGION = BATCH + NS * L

_mesh = plsc.VectorSubcoreMesh(core_axis_name="c", subcore_axis_name="s")


def _iota():
    return lax.iota(jnp.int32, L)


@functools.partial(
    pl.kernel,
    mesh=_mesh,
    out_type=(
        jax.ShapeDtypeStruct((NC, REGION, W), jnp.float32),
        jax.ShapeDtypeStruct((NC, NS, 8, 128), jnp.int32),
    ),
    compiler_params=pltpu.CompilerParams(
        use_tc_tiling_on_sc=True, needs_layout_passes=False
    ),
    scratch_types=[
        pltpu.VMEM((NBUF,), jnp.int32),
        pltpu.VMEM((NBUF,), jnp.int32),
        pltpu.VMEM((NBUF,), jnp.int32),
        pltpu.VMEM((3, DIM, 128), jnp.float32),
        pltpu.VMEM((2, L, W), jnp.float32),
        pltpu.VMEM((8, 128), jnp.int32),
        pltpu.SMEM((8,), jnp.int32),
        pltpu.SemaphoreType.DMA((4,)),
    ],
)
def _k1(table_t, idx_hbm, rows_out, meta_out, idx_all, mA, mB, slabs, ext,
        meta_v, scnt, sems):
    c = lax.axis_index("c")
    s = lax.axis_index("s")
    wid = s * NC + c
    lo = wid * STRIPE
    hi = lo + STRIPE
    slab0 = lo >> 7

    @pl.when(s == 0)
    def _():
        scnt[0] = jnp.int32(0)

    plsc.subcore_barrier()

    pltpu.sync_copy(idx_hbm, idx_all.at[pl.ds(0, BATCH)])

    def scan_body(g, off):
        v16 = idx_all[pl.ds(g * L, L)]
        m = (v16 >= lo) & (v16 < hi)
        mi = jnp.where(m, jnp.int32(1), jnp.int32(0))
        incl = plsc.cumsum(mi)
        pos16 = g * L + _iota()
        tgt = jnp.where(m, off + incl - mi, jnp.int32(TRASH))
        plsc.store_scatter(mA, [tgt], pos16)
        return off + incl[L - 1]

    cnt = lax.fori_loop(0, BATCH // L, scan_body, jnp.int32(0))
    gmax = (cnt + L - 1) >> 4

    pcnt = (cnt + L - 1) & ~(L - 1)
    start = plsc.fetch_and_add(scnt.at[0], pcnt, subcore_id=0)

    def radix_pass(src, dst, shift):
        def outer(b, off):
            def body(g, off):
                e16 = g * L + _iota()
                valid = e16 < cnt
                p16 = src[pl.ds(g * L, L)]
                v16 = plsc.load_gather(idx_all, [jnp.clip(p16, 0, NBUF - 1)])
                key = ((v16 >> 7) - slab0) >> shift
                m = ((key & 15) == b) & valid
                mi = jnp.where(m, jnp.int32(1), jnp.int32(0))
                incl = plsc.cumsum(mi)
                tgt = jnp.where(m, off + incl - mi, jnp.int32(TRASH))
                plsc.store_scatter(dst, [tgt], p16)
                return off + incl[L - 1]
            return lax.fori_loop(0, gmax, body, off)
        lax.fori_loop(0, 16, outer, jnp.int32(0))

    radix_pass(mA, mB, 0)
    radix_pass(mB, mA, 4)

    def runs_body(g, r):
        e16 = g * L + _iota()
        valid = e16 < cnt
        p16 = mA[pl.ds(g * L, L)]
        v16 = plsc.load_gather(idx_all, [jnp.clip(p16, 0, NBUF - 1)])
        pp = plsc.load_gather(mA, [jnp.clip(e16 - 1, 0, NBUF - 1)])
        vp = plsc.load_gather(idx_all, [jnp.clip(pp, 0, NBUF - 1)])
        ch = (((v16 >> 7) != (vp >> 7)) | (e16 == 0)) & valid
        mi = jnp.where(ch, jnp.int32(1), jnp.int32(0))
        incl = plsc.cumsum(mi)
        tgt = jnp.where(ch, r + incl - mi, jnp.int32(TRASH))
        plsc.store_scatter(mB, [tgt], e16)
        return r + incl[L - 1]

    nrun = lax.fori_loop(0, gmax, runs_body, jnp.int32(0))
    sent_tgt = jnp.where(_iota() == 0, nrun, jnp.int32(TRASH))
    plsc.store_scatter(mB, [sent_tgt], jnp.zeros((L,), jnp.int32) + cnt)

    def run_start(r):
        return mB[pl.ds(jnp.clip(r, 0, NBUF - L), L)][0]

    def elem_vocab(e):
        p = mA[pl.ds(jnp.clip(e, 0, NBUF - L), L)][0]
        v = idx_all[pl.ds(jnp.clip(p, 0, NBUF - L), L)][0]
        return p, v

    def fire_slab(r):
        _, v = elem_vocab(run_start(r))
        sb = v >> 7
        pltpu.async_copy(
            table_t.at[:, pl.ds(pl.multiple_of(sb * 128, 128), 128)],
            slabs.at[lax.rem(r, 3)],
            sems.at[lax.rem(r, 3)],
        )

    def drain_slab(r):
        pltpu.make_async_copy(
            table_t.at[:, pl.ds(0, 128)],
            slabs.at[lax.rem(r, 3)],
            sems.at[lax.rem(r, 3)],
        ).wait()

    @pl.when(nrun > 0)
    def _():
        fire_slab(0)

    @pl.when(nrun > 1)
    def _():
        fire_slab(1)

    def flush_wait():
        pltpu.make_async_copy(
            rows_out.at[c, pl.ds(0, L), :], ext.at[0], sems.at[3]
        ).wait()

    def per_run(r, carry):
        w, fc = carry
        drain_slab(r)

        @pl.when(r + 2 < nrun)
        def _():
            fire_slab(r + 2)

        e0 = run_start(r)
        e1 = run_start(r + 1)
        ring = lax.rem(r, 3)

        def per_elem(e, carry):
            w, fc = carry
            p, v = elem_vocab(e)
            col = v & 127
            buf = lax.rem(w >> 4, 2)
            slot = w & 15

            @pl.when((slot == 0) & (fc >= 2))
            def _():
                flush_wait()

            for k in range(4):
                vals = plsc.load_gather(
                    slabs,
                    [ring + 0 * _iota(), k * L + _iota(), col + 0 * _iota()],
                )
                ext[buf, slot, pl.ds(k * L, L)] = vals
            pf = plsc.bitcast(jnp.zeros((L,), jnp.int32) + p, jnp.float32)
            rtgt = jnp.zeros((L,), jnp.int32) + slot
            ctgt = jnp.where(_iota() == 0, jnp.int32(DIM), jnp.int32(W - 1))
            plsc.store_scatter(ext.at[buf], [rtgt, ctgt], pf)

            @pl.when(w == 0)
            def _():
                first = [ext[0, 0, pl.ds(k * L, L)] for k in range(4)]
                for bb in range(2):
                    for sl in range(L):
                        for k in range(4):
                            ext[bb, sl, pl.ds(k * L, L)] = first[k]
                        rt = jnp.zeros((L,), jnp.int32) + sl
                        plsc.store_scatter(ext.at[bb], [rt, ctgt], pf)

            @pl.when(slot == 15)
            def _():
                dst = pl.multiple_of(start + (w & ~15), L)
                pltpu.async_copy(
                    ext.at[buf],
                    rows_out.at[c, pl.ds(dst, L), :],
                    sems.at[3],
                )

            fc = jnp.where(slot == 15, fc + 1, fc)
            return (w + 1, fc)

        return lax.fori_loop(e0, e1, per_elem, (w, fc))

    w, fc = lax.fori_loop(0, nrun, per_run, (jnp.int32(0), jnp.int32(0)))

    @pl.when((w & 15) != 0)
    def _():
        dst = pl.multiple_of(start + (w & ~15), L)
        pltpu.async_copy(
            ext.at[lax.rem(w >> 4, 2)],
            rows_out.at[c, pl.ds(dst, L), :],
            sems.at[3],
        )

    nflush = fc + jnp.where((w & 15) != 0, jnp.int32(1), jnp.int32(0))
    ndrain = jnp.minimum(nflush, jnp.int32(2))

    def drain_body(i, z):
        flush_wait()
        return z

    lax.fori_loop(0, ndrain, drain_body, jnp.int32(0))

    mv = (jnp.where(_iota() == 0, start, 0)
          + jnp.where(_iota() == 1, pcnt, 0))
    meta_v[0, pl.ds(0, L)] = mv
    pltpu.sync_copy(meta_v, meta_out.at[c, s])


@functools.partial(
    pl.kernel,
    mesh=_mesh,
    out_type=jax.ShapeDtypeStruct((BATCH, DIM), jnp.float32),
    compiler_params=pltpu.CompilerParams(
        use_tc_tiling_on_sc=False, needs_layout_passes=False
    ),
    scratch_types=[
        pltpu.VMEM((8, 128), jnp.int32),
        pltpu.VMEM((4, L, W), jnp.float32),
        pltpu.VMEM((4, L, DIM), jnp.float32),
        pltpu.VMEM((4, L), jnp.int32),
        pltpu.SemaphoreType.DMA((8,)),
    ],
)
def _k2(rows_in, meta_in, out_hbm, meta_v, buf, cbuf, pos_v, sems):
    c = lax.axis_index("c")
    s = lax.axis_index("s")
    pltpu.sync_copy(meta_in.at[c, s], meta_v)
    head = meta_v[0, pl.ds(0, L)]
    start = head[0]
    pcnt = head[1]
    nq = pcnt >> 4

    def fire(q):
        off = pl.multiple_of(start + q * L, L)
        pltpu.async_copy(
            rows_in.at[c, pl.ds(off, L), :], buf.at[lax.rem(q, 4)],
            sems.at[lax.rem(q, 4)],
        )

    def drain_read(q):
        pltpu.make_async_copy(
            rows_in.at[c, pl.ds(0, L), :], buf.at[lax.rem(q, 4)],
            sems.at[lax.rem(q, 4)],
        ).wait()

    def drain_scatter(q):
        pltpu.make_async_copy(
            cbuf.at[lax.rem(q, 4)],
            out_hbm.at[pos_v.at[lax.rem(q, 4)]],
            sems.at[4 + lax.rem(q, 4)],
        ).wait()

    @pl.when(nq > 0)
    def _():
        fire(0)

    @pl.when(nq > 1)
    def _():
        fire(1)

    def per_chunk(q, z):
        drain_read(q)
        bq = lax.rem(q, 4)
        pf = plsc.load_gather(
            buf, [bq + 0 * _iota(), _iota(), jnp.int32(DIM) + 0 * _iota()]
        )
        pos_v[bq, pl.ds(0, L)] = plsc.bitcast(pf, jnp.int32)
        for l in range(L):
            for k in range(4):
                cbuf[bq, l, pl.ds(k * L, L)] = buf[bq, l, pl.ds(k * L, L)]
        pltpu.async_copy(
            cbuf.at[bq],
            out_hbm.at[pos_v.at[bq]],
            sems.at[4 + bq],
        )

        @pl.when(q + 2 < nq)
        def _():
            @pl.when(q >= 2)
            def _():
                drain_scatter(q - 2)
            fire(q + 2)

        return z

    lax.fori_loop(0, nq, per_chunk, jnp.int32(0))

    def tail_drain(i, z):
        drain_scatter(i)
        return z

    lax.fori_loop(jnp.maximum(nq - 4, 0), nq, tail_drain, jnp.int32(0))


def kernel(x, table):
    idx = x.reshape(BATCH).astype(jnp.int32)
    rows, meta = _k1(table.T, idx)
    return _k2(rows, meta)


# speedup vs baseline: 2.9908x; 1.1207x over previous
import functools

import jax
import jax.numpy as jnp
from jax import lax
from jax.experimental import pallas as pl
from jax.experimental.pallas import tpu as pltpu
from jax.experimental.pallas import tpu_sc as plsc

VOCAB = 1000000
DIM = 64
BATCH = 16384
W = 80

_info = plsc.get_sparse_core_info()
NC = _info.num_cores
NS = _info.num_subcores
NW = NC * NS
L = 16
STRIPE = VOCAB // NW
NBUF = BATCH + 32
TRASH = BATCH
REGION = BATCH + NS * L

_mesh = plsc.VectorSubcoreMesh(core_axis_name="c", subcore_axis_name="s")


def _iota():
    return lax.iota(jnp.int32, L)


@functools.partial(
    pl.kernel,
    mesh=_mesh,
    out_type=(
        jax.ShapeDtypeStruct((NC, REGION, W), jnp.float32),
        jax.ShapeDtypeStruct((NC, NS, 8, 128), jnp.int32),
    ),
    compiler_params=pltpu.CompilerParams(
        use_tc_tiling_on_sc=True, needs_layout_passes=False
    ),
    scratch_types=[
        pltpu.VMEM((NBUF,), jnp.int32),
        pltpu.VMEM((NBUF,), jnp.int32),
        pltpu.VMEM((NBUF,), jnp.int32),
        pltpu.VMEM((3, DIM, 128), jnp.float32),
        pltpu.VMEM((2, L, W), jnp.float32),
        pltpu.VMEM((8, 128), jnp.int32),
        pltpu.SMEM((8,), jnp.int32),
        pltpu.SemaphoreType.DMA((4,)),
    ],
)
def _k1(table_t, idx_hbm, rows_out, meta_out, idx_all, mA, mB, slabs, ext,
        meta_v, scnt, sems):
    c = lax.axis_index("c")
    s = lax.axis_index("s")
    wid = s * NC + c
    lo = wid * STRIPE
    hi = lo + STRIPE
    slab0 = lo >> 7

    @pl.when(s == 0)
    def _():
        scnt[0] = jnp.int32(0)

    plsc.subcore_barrier()

    pltpu.sync_copy(idx_hbm, idx_all.at[pl.ds(0, BATCH)])

    def scan_body(g, off):
        off0 = off
        for h in range(2):
            base = g * 2 * L + h * L
            v16 = idx_all[pl.ds(base, L)]
            m = (v16 >= lo) & (v16 < hi)
            mi = jnp.where(m, jnp.int32(1), jnp.int32(0))
            incl = plsc.cumsum(mi)
            pos16 = base + _iota()
            tgt = jnp.where(m, off0 + incl - mi, jnp.int32(TRASH))
            plsc.store_scatter(mA, [tgt], pos16)
            off0 = off0 + incl[L - 1]
        return off0

    cnt = lax.fori_loop(0, BATCH // (2 * L), scan_body, jnp.int32(0))
    gmax = (cnt + L - 1) >> 4

    pcnt = (cnt + L - 1) & ~(L - 1)
    start = plsc.fetch_and_add(scnt.at[0], pcnt, subcore_id=0)

    gmax2 = (cnt + 2 * L - 1) >> 5

    def radix_pass(src, dst, shift):
        def outer(b, off):
            def body(g, off):
                off0 = off
                for h in range(2):
                    base = g * 2 * L + h * L
                    e16 = base + _iota()
                    valid = e16 < cnt
                    p16 = src[pl.ds(base, L)]
                    v16 = plsc.load_gather(
                        idx_all, [jnp.clip(p16, 0, NBUF - 1)]
                    )
                    key = ((v16 >> 7) - slab0) >> shift
                    m = ((key & 15) == b) & valid
                    mi = jnp.where(m, jnp.int32(1), jnp.int32(0))
                    incl = plsc.cumsum(mi)
                    tgt = jnp.where(m, off0 + incl - mi, jnp.int32(TRASH))
                    plsc.store_scatter(dst, [tgt], p16)
                    off0 = off0 + incl[L - 1]
                return off0
            return lax.fori_loop(0, gmax2, body, off)
        lax.fori_loop(0, 16, outer, jnp.int32(0))

    radix_pass(mA, mB, 0)
    radix_pass(mB, mA, 4)

    def runs_body(g, r):
        e16 = g * L + _iota()
        valid = e16 < cnt
        p16 = mA[pl.ds(g * L, L)]
        v16 = plsc.load_gather(idx_all, [jnp.clip(p16, 0, NBUF - 1)])
        pp = plsc.load_gather(mA, [jnp.clip(e16 - 1, 0, NBUF - 1)])
        vp = plsc.load_gather(idx_all, [jnp.clip(pp, 0, NBUF - 1)])
        ch = (((v16 >> 7) != (vp >> 7)) | (e16 == 0)) & valid
        mi = jnp.where(ch, jnp.int32(1), jnp.int32(0))
        incl = plsc.cumsum(mi)
        tgt = jnp.where(ch, r + incl - mi, jnp.int32(TRASH))
        plsc.store_scatter(mB, [tgt], e16)
        return r + incl[L - 1]

    nrun = lax.fori_loop(0, gmax, runs_body, jnp.int32(0))
    sent_tgt = jnp.where(_iota() == 0, nrun, jnp.int32(TRASH))
    plsc.store_scatter(mB, [sent_tgt], jnp.zeros((L,), jnp.int32) + cnt)

    def run_start(r):
        return mB[pl.ds(jnp.clip(r, 0, NBUF - L), L)][0]

    def elem_vocab(e):
        p = mA[pl.ds(jnp.clip(e, 0, NBUF - L), L)][0]
        v = idx_all[pl.ds(jnp.clip(p, 0, NBUF - L), L)][0]
        return p, v

    def fire_slab(r):
        _, v = elem_vocab(run_start(r))
        sb = v >> 7
        pltpu.async_copy(
            table_t.at[:, pl.ds(pl.multiple_of(sb * 128, 128), 128)],
            slabs.at[lax.rem(r, 3)],
            sems.at[lax.rem(r, 3)],
        )

    def drain_slab(r):
        pltpu.make_async_copy(
            table_t.at[:, pl.ds(0, 128)],
            slabs.at[lax.rem(r, 3)],
            sems.at[lax.rem(r, 3)],
        ).wait()

    @pl.when(nrun > 0)
    def _():
        fire_slab(0)

    @pl.when(nrun > 1)
    def _():
        fire_slab(1)

    def flush_wait():
        pltpu.make_async_copy(
            rows_out.at[c, pl.ds(0, L), :], ext.at[0], sems.at[3]
        ).wait()

    def per_run(r, carry):
        w, fc = carry
        drain_slab(r)

        @pl.when(r + 2 < nrun)
        def _():
            fire_slab(r + 2)

        e0 = run_start(r)
        e1 = run_start(r + 1)
        ring = lax.rem(r, 3)

        def per_elem(e, carry):
            w, fc = carry
            p, v = elem_vocab(e)
            col = v & 127
            buf = lax.rem(w >> 4, 2)
            slot = w & 15

            @pl.when((slot == 0) & (fc >= 2))
            def _():
                flush_wait()

            for k in range(4):
                vals = plsc.load_gather(
                    slabs,
                    [ring + 0 * _iota(), k * L + _iota(), col + 0 * _iota()],
                )
                ext[buf, slot, pl.ds(k * L, L)] = vals
            pf = plsc.bitcast(jnp.zeros((L,), jnp.int32) + p, jnp.float32)
            rtgt = jnp.zeros((L,), jnp.int32) + slot
            ctgt = jnp.where(_iota() == 0, jnp.int32(DIM), jnp.int32(W - 1))
            plsc.store_scatter(ext.at[buf], [rtgt, ctgt], pf)

            @pl.when(w == 0)
            def _():
                first = [ext[0, 0, pl.ds(k * L, L)] for k in range(4)]
                for bb in range(2):
                    for sl in range(L):
                        for k in range(4):
                            ext[bb, sl, pl.ds(k * L, L)] = first[k]
                        rt = jnp.zeros((L,), jnp.int32) + sl
                        plsc.store_scatter(ext.at[bb], [rt, ctgt], pf)

            @pl.when(slot == 15)
            def _():
                dst = pl.multiple_of(start + (w & ~15), L)
                pltpu.async_copy(
                    ext.at[buf],
                    rows_out.at[c, pl.ds(dst, L), :],
                    sems.at[3],
                )

            fc = jnp.where(slot == 15, fc + 1, fc)
            return (w + 1, fc)

        return lax.fori_loop(e0, e1, per_elem, (w, fc))

    w, fc = lax.fori_loop(0, nrun, per_run, (jnp.int32(0), jnp.int32(0)))

    @pl.when((w & 15) != 0)
    def _():
        dst = pl.multiple_of(start + (w & ~15), L)
        pltpu.async_copy(
            ext.at[lax.rem(w >> 4, 2)],
            rows_out.at[c, pl.ds(dst, L), :],
            sems.at[3],
        )

    nflush = fc + jnp.where((w & 15) != 0, jnp.int32(1), jnp.int32(0))
    ndrain = jnp.minimum(nflush, jnp.int32(2))

    def drain_body(i, z):
        flush_wait()
        return z

    lax.fori_loop(0, ndrain, drain_body, jnp.int32(0))

    mv = (jnp.where(_iota() == 0, start, 0)
          + jnp.where(_iota() == 1, pcnt, 0))
    meta_v[0, pl.ds(0, L)] = mv
    pltpu.sync_copy(meta_v, meta_out.at[c, s])


@functools.partial(
    pl.kernel,
    mesh=_mesh,
    out_type=jax.ShapeDtypeStruct((BATCH, 128), jnp.float32),
    compiler_params=pltpu.CompilerParams(
        use_tc_tiling_on_sc=True, needs_layout_passes=False
    ),
    scratch_types=[
        pltpu.VMEM((8, 128), jnp.int32),
        pltpu.VMEM((4, L, W), jnp.float32),
        pltpu.VMEM((4, L, 128), jnp.float32),
        pltpu.VMEM((4, L), jnp.int32),
        pltpu.SemaphoreType.DMA((8,)),
    ],
)
def _k2(rows_in, meta_in, out_hbm, meta_v, buf, cbuf, pos_v, sems):
    c = lax.axis_index("c")
    s = lax.axis_index("s")
    pltpu.sync_copy(meta_in.at[c, s], meta_v)
    head = meta_v[0, pl.ds(0, L)]
    start = head[0]
    pcnt = head[1]
    nq = pcnt >> 4

    def fire(q):
        off = pl.multiple_of(start + q * L, L)
        pltpu.async_copy(
            rows_in.at[c, pl.ds(off, L), :], buf.at[lax.rem(q, 4)],
            sems.at[lax.rem(q, 4)],
        )

    def drain_read(q):
        pltpu.make_async_copy(
            rows_in.at[c, pl.ds(0, L), :], buf.at[lax.rem(q, 4)],
            sems.at[lax.rem(q, 4)],
        ).wait()

    def drain_scatter(q):
        pltpu.make_async_copy(
            cbuf.at[lax.rem(q, 4)],
            out_hbm.at[pos_v.at[lax.rem(q, 4)]],
            sems.at[4 + lax.rem(q, 4)],
        ).wait()

    @pl.when(nq > 0)
    def _():
        fire(0)

    @pl.when(nq > 1)
    def _():
        fire(1)

    def per_chunk(q, z):
        drain_read(q)
        bq = lax.rem(q, 4)
        pf = plsc.load_gather(
            buf, [bq + 0 * _iota(), _iota(), jnp.int32(DIM) + 0 * _iota()]
        )
        pos_v[bq, pl.ds(0, L)] = plsc.bitcast(pf, jnp.int32)
        for l in range(L):
            for k in range(4):
                cbuf[bq, l, pl.ds(k * L, L)] = buf[bq, l, pl.ds(k * L, L)]
        pltpu.async_copy(
            cbuf.at[bq],
            out_hbm.at[pos_v.at[bq]],
            sems.at[4 + bq],
        )

        @pl.when(q + 2 < nq)
        def _():
            @pl.when(q >= 2)
            def _():
                drain_scatter(q - 2)
            fire(q + 2)

        return z

    lax.fori_loop(0, nq, per_chunk, jnp.int32(0))

    def tail_drain(i, z):
        drain_scatter(i)
        return z

    lax.fori_loop(jnp.maximum(nq - 4, 0), nq, tail_drain, jnp.int32(0))


def kernel(x, table):
    idx = x.reshape(BATCH).astype(jnp.int32)
    rows, meta = _k1(table.T, idx)
    return _k2(rows, meta)[:, :DIM]


# speedup vs baseline: 3.7322x; 1.2479x over previous
import functools

import jax
import jax.numpy as jnp
from jax import lax
from jax.experimental import pallas as pl
from jax.experimental.pallas import tpu as pltpu
from jax.experimental.pallas import tpu_sc as plsc

VOCAB = 1000000
DIM = 64
BATCH = 16384
W = 80

_info = plsc.get_sparse_core_info()
NC = _info.num_cores
NS = _info.num_subcores
NW = NC * NS
L = 16
STRIPE = VOCAB // NW
NBUF = BATCH + 32
TRASH = BATCH
REGION = BATCH + NS * L

_mesh = plsc.VectorSubcoreMesh(core_axis_name="c", subcore_axis_name="s")


def _iota():
    return lax.iota(jnp.int32, L)


@functools.partial(
    pl.kernel,
    mesh=_mesh,
    out_type=(
        jax.ShapeDtypeStruct((NC, REGION, W), jnp.float32),
        jax.ShapeDtypeStruct((NC, NS, 8, 128), jnp.int32),
    ),
    compiler_params=pltpu.CompilerParams(
        use_tc_tiling_on_sc=True, needs_layout_passes=False
    ),
    scratch_types=[
        pltpu.VMEM((NBUF,), jnp.int32),
        pltpu.VMEM((NBUF,), jnp.int32),
        pltpu.VMEM((NBUF,), jnp.int32),
        pltpu.VMEM((6, DIM, 128), jnp.float32),
        pltpu.VMEM((2, L, W), jnp.float32),
        pltpu.VMEM((8, 128), jnp.int32),
        pltpu.SMEM((8,), jnp.int32),
        pltpu.SemaphoreType.DMA((8,)),
    ],
)
def _k1(table_t, idx_hbm, rows_out, meta_out, idx_all, mA, mB, slabs, ext,
        meta_v, scnt, sems):
    c = lax.axis_index("c")
    s = lax.axis_index("s")
    wid = s * NC + c
    lo = wid * STRIPE
    hi = lo + STRIPE
    slab0 = lo >> 7

    @pl.when(s == 0)
    def _():
        scnt[0] = jnp.int32(0)

    plsc.subcore_barrier()

    pltpu.sync_copy(idx_hbm, idx_all.at[pl.ds(0, BATCH)])

    def scan_body(g, off):
        off0 = off
        for h in range(2):
            base = g * 2 * L + h * L
            v16 = idx_all[pl.ds(base, L)]
            m = (v16 >= lo) & (v16 < hi)
            mi = jnp.where(m, jnp.int32(1), jnp.int32(0))
            incl = plsc.cumsum(mi)
            pos16 = base + _iota()
            tgt = jnp.where(m, off0 + incl - mi, jnp.int32(TRASH))
            plsc.store_scatter(mA, [tgt], pos16)
            off0 = off0 + incl[L - 1]
        return off0

    cnt = lax.fori_loop(0, BATCH // (2 * L), scan_body, jnp.int32(0))
    gmax = (cnt + L - 1) >> 4

    pcnt = (cnt + L - 1) & ~(L - 1)
    start = plsc.fetch_and_add(scnt.at[0], pcnt, subcore_id=0)

    gmax2 = (cnt + 2 * L - 1) >> 5

    def radix_pass(src, dst, shift):
        def outer(b, off):
            def body(g, off):
                off0 = off
                for h in range(2):
                    base = g * 2 * L + h * L
                    e16 = base + _iota()
                    valid = e16 < cnt
                    p16 = src[pl.ds(base, L)]
                    v16 = plsc.load_gather(
                        idx_all, [jnp.clip(p16, 0, NBUF - 1)]
                    )
                    key = ((v16 >> 7) - slab0) >> shift
                    m = ((key & 15) == b) & valid
                    mi = jnp.where(m, jnp.int32(1), jnp.int32(0))
                    incl = plsc.cumsum(mi)
                    tgt = jnp.where(m, off0 + incl - mi, jnp.int32(TRASH))
                    plsc.store_scatter(dst, [tgt], p16)
                    off0 = off0 + incl[L - 1]
                return off0
            return lax.fori_loop(0, gmax2, body, off)
        lax.fori_loop(0, 16, outer, jnp.int32(0))

    radix_pass(mA, mB, 0)
    radix_pass(mB, mA, 4)

    def runs_body(g, r):
        e16 = g * L + _iota()
        valid = e16 < cnt
        p16 = mA[pl.ds(g * L, L)]
        v16 = plsc.load_gather(idx_all, [jnp.clip(p16, 0, NBUF - 1)])
        pp = plsc.load_gather(mA, [jnp.clip(e16 - 1, 0, NBUF - 1)])
        vp = plsc.load_gather(idx_all, [jnp.clip(pp, 0, NBUF - 1)])
        ch = (((v16 >> 7) != (vp >> 7)) | (e16 == 0)) & valid
        mi = jnp.where(ch, jnp.int32(1), jnp.int32(0))
        incl = plsc.cumsum(mi)
        tgt = jnp.where(ch, r + incl - mi, jnp.int32(TRASH))
        plsc.store_scatter(mB, [tgt], e16)
        return r + incl[L - 1]

    nrun = lax.fori_loop(0, gmax, runs_body, jnp.int32(0))
    sent_tgt = jnp.where(_iota() == 0, nrun, jnp.int32(TRASH))
    plsc.store_scatter(mB, [sent_tgt], jnp.zeros((L,), jnp.int32) + cnt)

    def run_start(r):
        return mB[pl.ds(jnp.clip(r, 0, NBUF - L), L)][0]

    def elem_vocab(e):
        p = mA[pl.ds(jnp.clip(e, 0, NBUF - L), L)][0]
        v = idx_all[pl.ds(jnp.clip(p, 0, NBUF - L), L)][0]
        return p, v

    def fire_slab(r):
        _, v = elem_vocab(run_start(r))
        sb = v >> 7
        pltpu.async_copy(
            table_t.at[:, pl.ds(pl.multiple_of(sb * 128, 128), 128)],
            slabs.at[lax.rem(r, 6)],
            sems.at[lax.rem(r, 6)],
        )

    def drain_slab(r):
        pltpu.make_async_copy(
            table_t.at[:, pl.ds(0, 128)],
            slabs.at[lax.rem(r, 6)],
            sems.at[lax.rem(r, 6)],
        ).wait()

    for rr in range(4):
        @pl.when(nrun > rr)
        def _(rr=rr):
            fire_slab(rr)

    def flush_wait():
        pltpu.make_async_copy(
            rows_out.at[c, pl.ds(0, L), :], ext.at[0], sems.at[6]
        ).wait()

    def per_run(r, carry):
        w, fc = carry
        drain_slab(r)

        @pl.when(r + 4 < nrun)
        def _():
            fire_slab(r + 4)

        e0 = run_start(r)
        e1 = run_start(r + 1)
        ring = lax.rem(r, 6)

        def per_elem(e, carry):
            w, fc = carry
            p, v = elem_vocab(e)
            col = v & 127
            buf = lax.rem(w >> 4, 2)
            slot = w & 15

            @pl.when((slot == 0) & (fc >= 2))
            def _():
                flush_wait()

            for k in range(4):
                vals = plsc.load_gather(
                    slabs,
                    [ring + 0 * _iota(), k * L + _iota(), col + 0 * _iota()],
                )
                ext[buf, slot, pl.ds(k * L, L)] = vals
            pf = plsc.bitcast(jnp.zeros((L,), jnp.int32) + p, jnp.float32)
            rtgt = jnp.zeros((L,), jnp.int32) + slot
            ctgt = jnp.where(_iota() == 0, jnp.int32(DIM), jnp.int32(W - 1))
            plsc.store_scatter(ext.at[buf], [rtgt, ctgt], pf)

            @pl.when(w == 0)
            def _():
                first = [ext[0, 0, pl.ds(k * L, L)] for k in range(4)]
                for bb in range(2):
                    for sl in range(L):
                        for k in range(4):
                            ext[bb, sl, pl.ds(k * L, L)] = first[k]
                        rt = jnp.zeros((L,), jnp.int32) + sl
                        plsc.store_scatter(ext.at[bb], [rt, ctgt], pf)

            @pl.when(slot == 15)
            def _():
                dst = pl.multiple_of(start + (w & ~15), L)
                pltpu.async_copy(
                    ext.at[buf],
                    rows_out.at[c, pl.ds(dst, L), :],
                    sems.at[6],
                )

            fc = jnp.where(slot == 15, fc + 1, fc)
            return (w + 1, fc)

        return lax.fori_loop(e0, e1, per_elem, (w, fc))

    w, fc = lax.fori_loop(0, nrun, per_run, (jnp.int32(0), jnp.int32(0)))

    @pl.when((w & 15) != 0)
    def _():
        dst = pl.multiple_of(start + (w & ~15), L)
        pltpu.async_copy(
            ext.at[lax.rem(w >> 4, 2)],
            rows_out.at[c, pl.ds(dst, L), :],
            sems.at[6],
        )

    nflush = fc + jnp.where((w & 15) != 0, jnp.int32(1), jnp.int32(0))
    ndrain = jnp.minimum(nflush, jnp.int32(2))

    def drain_body(i, z):
        flush_wait()
        return z

    lax.fori_loop(0, ndrain, drain_body, jnp.int32(0))

    mv = (jnp.where(_iota() == 0, start, 0)
          + jnp.where(_iota() == 1, pcnt, 0))
    meta_v[0, pl.ds(0, L)] = mv
    pltpu.sync_copy(meta_v, meta_out.at[c, s])


@functools.partial(
    pl.kernel,
    mesh=_mesh,
    out_type=jax.ShapeDtypeStruct((BATCH, 128), jnp.float32),
    compiler_params=pltpu.CompilerParams(
        use_tc_tiling_on_sc=True, needs_layout_passes=False
    ),
    scratch_types=[
        pltpu.VMEM((8, 128), jnp.int32),
        pltpu.VMEM((4, L, W), jnp.float32),
        pltpu.VMEM((4, L, 128), jnp.float32),
        pltpu.VMEM((4, L), jnp.int32),
        pltpu.SemaphoreType.DMA((8,)),
    ],
)
def _k2(rows_in, meta_in, out_hbm, meta_v, buf, cbuf, pos_v, sems):
    c = lax.axis_index("c")
    s = lax.axis_index("s")
    pltpu.sync_copy(meta_in.at[c, s], meta_v)
    head = meta_v[0, pl.ds(0, L)]
    start = head[0]
    pcnt = head[1]
    nq = pcnt >> 4

    def fire(q):
        off = pl.multiple_of(start + q * L, L)
        pltpu.async_copy(
            rows_in.at[c, pl.ds(off, L), :], buf.at[lax.rem(q, 4)],
            sems.at[lax.rem(q, 4)],
        )

    def drain_read(q):
        pltpu.make_async_copy(
            rows_in.at[c, pl.ds(0, L), :], buf.at[lax.rem(q, 4)],
            sems.at[lax.rem(q, 4)],
        ).wait()

    def drain_scatter(q):
        pltpu.make_async_copy(
            cbuf.at[lax.rem(q, 4)],
            out_hbm.at[pos_v.at[lax.rem(q, 4)]],
            sems.at[4 + lax.rem(q, 4)],
        ).wait()

    @pl.when(nq > 0)
    def _():
        fire(0)

    @pl.when(nq > 1)
    def _():
        fire(1)

    def per_chunk(q, z):
        drain_read(q)
        bq = lax.rem(q, 4)
        pf = plsc.load_gather(
            buf, [bq + 0 * _iota(), _iota(), jnp.int32(DIM) + 0 * _iota()]
        )
        pos_v[bq, pl.ds(0, L)] = plsc.bitcast(pf, jnp.int32)
        for l in range(L):
            for k in range(4):
                cbuf[bq, l, pl.ds(k * L, L)] = buf[bq, l, pl.ds(k * L, L)]
        pltpu.async_copy(
            cbuf.at[bq],
            out_hbm.at[pos_v.at[bq]],
            sems.at[4 + bq],
        )

        @pl.when(q + 2 < nq)
        def _():
            @pl.when(q >= 2)
            def _():
                drain_scatter(q - 2)
            fire(q + 2)

        return z

    lax.fori_loop(0, nq, per_chunk, jnp.int32(0))

    def tail_drain(i, z):
        drain_scatter(i)
        return z

    lax.fori_loop(jnp.maximum(nq - 4, 0), nq, tail_drain, jnp.int32(0))


def kernel(x, table):
    idx = x.reshape(BATCH).astype(jnp.int32)
    rows, meta = _k1(table.T, idx)
    return _k2(rows, meta)[:, :DIM]


# speedup vs baseline: 3.8502x; 1.0316x over previous
import functools

import jax
import jax.numpy as jnp
from jax import lax
from jax.experimental import pallas as pl
from jax.experimental.pallas import tpu as pltpu
from jax.experimental.pallas import tpu_sc as plsc

VOCAB = 1000000
DIM = 64
BATCH = 16384
W = 80

_info = plsc.get_sparse_core_info()
NC = _info.num_cores
NS = _info.num_subcores
NW = NC * NS
L = 16
STRIPE = VOCAB // NW
NBUF = BATCH + 32
TRASH = BATCH
REGION = BATCH + NS * L

_mesh = plsc.VectorSubcoreMesh(core_axis_name="c", subcore_axis_name="s")


def _iota():
    return lax.iota(jnp.int32, L)


@functools.partial(
    pl.kernel,
    mesh=_mesh,
    out_type=(
        jax.ShapeDtypeStruct((NC, REGION, W), jnp.float32),
        jax.ShapeDtypeStruct((NC, NS, 8, 128), jnp.int32),
    ),
    compiler_params=pltpu.CompilerParams(
        use_tc_tiling_on_sc=True, needs_layout_passes=False
    ),
    scratch_types=[
        pltpu.VMEM((NBUF,), jnp.int32),
        pltpu.VMEM((NBUF,), jnp.int32),
        pltpu.VMEM((NBUF,), jnp.int32),
        pltpu.VMEM((8, DIM, 128), jnp.float32),
        pltpu.VMEM((2, L, W), jnp.float32),
        pltpu.VMEM((8, 128), jnp.int32),
        pltpu.SMEM((8,), jnp.int32),
        pltpu.SemaphoreType.DMA((9,)),
    ],
)
def _k1(table_t, idx_hbm, rows_out, meta_out, idx_all, mA, mB, slabs, ext,
        meta_v, scnt, sems):
    c = lax.axis_index("c")
    s = lax.axis_index("s")
    wid = s * NC + c
    lo = wid * STRIPE
    hi = lo + STRIPE
    slab0 = lo >> 7

    @pl.when(s == 0)
    def _():
        scnt[0] = jnp.int32(0)

    plsc.subcore_barrier()

    pltpu.sync_copy(idx_hbm, idx_all.at[pl.ds(0, BATCH)])

    def scan_body(g, off):
        off0 = off
        for h in range(2):
            base = g * 2 * L + h * L
            v16 = idx_all[pl.ds(base, L)]
            m = (v16 >= lo) & (v16 < hi)
            mi = jnp.where(m, jnp.int32(1), jnp.int32(0))
            incl = plsc.cumsum(mi)
            pos16 = base + _iota()
            tgt = jnp.where(m, off0 + incl - mi, jnp.int32(TRASH))
            plsc.store_scatter(mA, [tgt], pos16)
            off0 = off0 + incl[L - 1]
        return off0

    cnt = lax.fori_loop(0, BATCH // (2 * L), scan_body, jnp.int32(0))
    gmax = (cnt + L - 1) >> 4

    pcnt = (cnt + L - 1) & ~(L - 1)
    start = plsc.fetch_and_add(scnt.at[0], pcnt, subcore_id=0)

    gmax2 = (cnt + 2 * L - 1) >> 5

    def radix_pass(src, dst, shift):
        def outer(b, off):
            def body(g, off):
                off0 = off
                for h in range(2):
                    base = g * 2 * L + h * L
                    e16 = base + _iota()
                    valid = e16 < cnt
                    p16 = src[pl.ds(base, L)]
                    v16 = plsc.load_gather(
                        idx_all, [jnp.clip(p16, 0, NBUF - 1)]
                    )
                    key = ((v16 >> 7) - slab0) >> shift
                    m = ((key & 15) == b) & valid
                    mi = jnp.where(m, jnp.int32(1), jnp.int32(0))
                    incl = plsc.cumsum(mi)
                    tgt = jnp.where(m, off0 + incl - mi, jnp.int32(TRASH))
                    plsc.store_scatter(dst, [tgt], p16)
                    off0 = off0 + incl[L - 1]
                return off0
            return lax.fori_loop(0, gmax2, body, off)
        lax.fori_loop(0, 16, outer, jnp.int32(0))

    radix_pass(mA, mB, 0)
    radix_pass(mB, mA, 4)

    def runs_body(g, r):
        e16 = g * L + _iota()
        valid = e16 < cnt
        p16 = mA[pl.ds(g * L, L)]
        v16 = plsc.load_gather(idx_all, [jnp.clip(p16, 0, NBUF - 1)])
        pp = plsc.load_gather(mA, [jnp.clip(e16 - 1, 0, NBUF - 1)])
        vp = plsc.load_gather(idx_all, [jnp.clip(pp, 0, NBUF - 1)])
        ch = (((v16 >> 7) != (vp >> 7)) | (e16 == 0)) & valid
        mi = jnp.where(ch, jnp.int32(1), jnp.int32(0))
        incl = plsc.cumsum(mi)
        tgt = jnp.where(ch, r + incl - mi, jnp.int32(TRASH))
        plsc.store_scatter(mB, [tgt], e16)
        return r + incl[L - 1]

    nrun = lax.fori_loop(0, gmax, runs_body, jnp.int32(0))
    sent_tgt = jnp.where(_iota() == 0, nrun, jnp.int32(TRASH))
    plsc.store_scatter(mB, [sent_tgt], jnp.zeros((L,), jnp.int32) + cnt)

    def run_start(r):
        return mB[pl.ds(jnp.clip(r, 0, NBUF - L), L)][0]

    def elem_vocab(e):
        p = mA[pl.ds(jnp.clip(e, 0, NBUF - L), L)][0]
        v = idx_all[pl.ds(jnp.clip(p, 0, NBUF - L), L)][0]
        return p, v

    def fire_slab(r):
        _, v = elem_vocab(run_start(r))
        sb = v >> 7
        pltpu.async_copy(
            table_t.at[:, pl.ds(pl.multiple_of(sb * 128, 128), 128)],
            slabs.at[lax.rem(r, 8)],
            sems.at[lax.rem(r, 8)],
        )

    def drain_slab(r):
        pltpu.make_async_copy(
            table_t.at[:, pl.ds(0, 128)],
            slabs.at[lax.rem(r, 8)],
            sems.at[lax.rem(r, 8)],
        ).wait()

    for rr in range(6):
        @pl.when(nrun > rr)
        def _(rr=rr):
            fire_slab(rr)

    def flush_wait():
        pltpu.make_async_copy(
            rows_out.at[c, pl.ds(0, L), :], ext.at[0], sems.at[8]
        ).wait()

    def per_run(r, carry):
        w, fc = carry
        drain_slab(r)

        @pl.when(r + 6 < nrun)
        def _():
            fire_slab(r + 6)

        e0 = run_start(r)
        e1 = run_start(r + 1)
        ring = lax.rem(r, 8)

        def per_elem(e, carry):
            w, fc = carry
            p, v = elem_vocab(e)
            col = v & 127
            buf = lax.rem(w >> 4, 2)
            slot = w & 15

            @pl.when((slot == 0) & (fc >= 2))
            def _():
                flush_wait()

            for k in range(4):
                vals = plsc.load_gather(
                    slabs,
                    [ring + 0 * _iota(), k * L + _iota(), col + 0 * _iota()],
                )
                ext[buf, slot, pl.ds(k * L, L)] = vals
            pf = plsc.bitcast(jnp.zeros((L,), jnp.int32) + p, jnp.float32)
            rtgt = jnp.zeros((L,), jnp.int32) + slot
            ctgt = jnp.where(_iota() == 0, jnp.int32(DIM), jnp.int32(W - 1))
            plsc.store_scatter(ext.at[buf], [rtgt, ctgt], pf)

            @pl.when(w == 0)
            def _():
                first = [ext[0, 0, pl.ds(k * L, L)] for k in range(4)]
                for bb in range(2):
                    for sl in range(L):
                        for k in range(4):
                            ext[bb, sl, pl.ds(k * L, L)] = first[k]
                        rt = jnp.zeros((L,), jnp.int32) + sl
                        plsc.store_scatter(ext.at[bb], [rt, ctgt], pf)

            @pl.when(slot == 15)
            def _():
                dst = pl.multiple_of(start + (w & ~15), L)
                pltpu.async_copy(
                    ext.at[buf],
                    rows_out.at[c, pl.ds(dst, L), :],
                    sems.at[8],
                )

            fc = jnp.where(slot == 15, fc + 1, fc)
            return (w + 1, fc)

        return lax.fori_loop(e0, e1, per_elem, (w, fc))

    w, fc = lax.fori_loop(0, nrun, per_run, (jnp.int32(0), jnp.int32(0)))

    @pl.when((w & 15) != 0)
    def _():
        dst = pl.multiple_of(start + (w & ~15), L)
        pltpu.async_copy(
            ext.at[lax.rem(w >> 4, 2)],
            rows_out.at[c, pl.ds(dst, L), :],
            sems.at[8],
        )

    nflush = fc + jnp.where((w & 15) != 0, jnp.int32(1), jnp.int32(0))
    ndrain = jnp.minimum(nflush, jnp.int32(2))

    def drain_body(i, z):
        flush_wait()
        return z

    lax.fori_loop(0, ndrain, drain_body, jnp.int32(0))

    mv = (jnp.where(_iota() == 0, start, 0)
          + jnp.where(_iota() == 1, pcnt, 0))
    meta_v[0, pl.ds(0, L)] = mv
    pltpu.sync_copy(meta_v, meta_out.at[c, s])


@functools.partial(
    pl.kernel,
    mesh=_mesh,
    out_type=jax.ShapeDtypeStruct((BATCH, 128), jnp.float32),
    compiler_params=pltpu.CompilerParams(
        use_tc_tiling_on_sc=True, needs_layout_passes=False
    ),
    scratch_types=[
        pltpu.VMEM((8, 128), jnp.int32),
        pltpu.VMEM((4, L, W), jnp.float32),
        pltpu.VMEM((4, L, 128), jnp.float32),
        pltpu.VMEM((4, L), jnp.int32),
        pltpu.SemaphoreType.DMA((8,)),
    ],
)
def _k2(rows_in, meta_in, out_hbm, meta_v, buf, cbuf, pos_v, sems):
    c = lax.axis_index("c")
    s = lax.axis_index("s")
    pltpu.sync_copy(meta_in.at[c, s], meta_v)
    head = meta_v[0, pl.ds(0, L)]
    start = head[0]
    pcnt = head[1]
    nq = pcnt >> 4

    def fire(q):
        off = pl.multiple_of(start + q * L, L)
        pltpu.async_copy(
            rows_in.at[c, pl.ds(off, L), :], buf.at[lax.rem(q, 4)],
            sems.at[lax.rem(q, 4)],
        )

    def drain_read(q):
        pltpu.make_async_copy(
            rows_in.at[c, pl.ds(0, L), :], buf.at[lax.rem(q, 4)],
            sems.at[lax.rem(q, 4)],
        ).wait()

    def drain_scatter(q):
        pltpu.make_async_copy(
            cbuf.at[lax.rem(q, 4)],
            out_hbm.at[pos_v.at[lax.rem(q, 4)]],
            sems.at[4 + lax.rem(q, 4)],
        ).wait()

    @pl.when(nq > 0)
    def _():
        fire(0)

    @pl.when(nq > 1)
    def _():
        fire(1)

    def per_chunk(q, z):
        drain_read(q)
        bq = lax.rem(q, 4)
        pf = plsc.load_gather(
            buf, [bq + 0 * _iota(), _iota(), jnp.int32(DIM) + 0 * _iota()]
        )
        pos_v[bq, pl.ds(0, L)] = plsc.bitcast(pf, jnp.int32)
        for l in range(L):
            for k in range(4):
                cbuf[bq, l, pl.ds(k * L, L)] = buf[bq, l, pl.ds(k * L, L)]
        pltpu.async_copy(
            cbuf.at[bq],
            out_hbm.at[pos_v.at[bq]],
            sems.at[4 + bq],
        )

        @pl.when(q + 2 < nq)
        def _():
            @pl.when(q >= 2)
            def _():
                drain_scatter(q - 2)
            fire(q + 2)

        return z

    lax.fori_loop(0, nq, per_chunk, jnp.int32(0))

    def tail_drain(i, z):
        drain_scatter(i)
        return z

    lax.fori_loop(jnp.maximum(nq - 4, 0), nq, tail_drain, jnp.int32(0))


def kernel(x, table):
    idx = x.reshape(BATCH).astype(jnp.int32)
    rows, meta = _k1(table.T, idx)
    return _k2(rows, meta)[:, :DIM]


# speedup vs baseline: 3.8577x; 1.0019x over previous
import functools

import jax
import jax.numpy as jnp
from jax import lax
from jax.experimental import pallas as pl
from jax.experimental.pallas import tpu as pltpu
from jax.experimental.pallas import tpu_sc as plsc

VOCAB = 1000000
DIM = 64
BATCH = 16384
W = 80

try:
    _info = plsc.get_sparse_core_info()
    NC = _info.num_cores
    NS = _info.num_subcores
except Exception:
    NC, NS = 2, 16
NW = NC * NS
L = 16
STRIPE = VOCAB // NW
NBUF = BATCH + 32
TRASH = BATCH
REGION = BATCH + NS * L

_mesh = plsc.VectorSubcoreMesh(core_axis_name="c", subcore_axis_name="s")


def _iota():
    return lax.iota(jnp.int32, L)


@functools.partial(
    pl.kernel,
    mesh=_mesh,
    out_type=(
        jax.ShapeDtypeStruct((NC, REGION, W), jnp.float32),
        jax.ShapeDtypeStruct((NC, NS, 8, 128), jnp.int32),
    ),
    compiler_params=pltpu.CompilerParams(
        use_tc_tiling_on_sc=True, needs_layout_passes=False
    ),
    scratch_types=[
        pltpu.VMEM((NBUF,), jnp.int32),
        pltpu.VMEM((NBUF,), jnp.int32),
        pltpu.VMEM((NBUF,), jnp.int32),
        pltpu.VMEM((8, DIM, 128), jnp.float32),
        pltpu.VMEM((2, L, W), jnp.float32),
        pltpu.VMEM((8, 128), jnp.int32),
        pltpu.SMEM((8,), jnp.int32),
        pltpu.SemaphoreType.DMA((9,)),
    ],
)
def _k1(table_t, idx_hbm, rows_out, meta_out, idx_all, mA, mB, slabs, ext,
        meta_v, scnt, sems):
    c = lax.axis_index("c")
    s = lax.axis_index("s")
    wid = s * NC + c
    lo = wid * STRIPE
    hi = lo + STRIPE
    slab0 = lo >> 7

    @pl.when(s == 0)
    def _():
        scnt[0] = jnp.int32(0)

    plsc.subcore_barrier()

    pltpu.sync_copy(idx_hbm, idx_all.at[pl.ds(0, BATCH)])

    def scan_body(g, off):
        off0 = off
        for h in range(2):
            base = g * 2 * L + h * L
            v16 = idx_all[pl.ds(base, L)]
            m = (v16 >= lo) & (v16 < hi)
            mi = jnp.where(m, jnp.int32(1), jnp.int32(0))
            incl = plsc.cumsum(mi)
            pos16 = base + _iota()
            tgt = jnp.where(m, off0 + incl - mi, jnp.int32(TRASH))
            plsc.store_scatter(mA, [tgt], pos16)
            off0 = off0 + incl[L - 1]
        return off0

    cnt = lax.fori_loop(0, BATCH // (2 * L), scan_body, jnp.int32(0))
    gmax = (cnt + L - 1) >> 4

    pcnt = (cnt + L - 1) & ~(L - 1)
    start = plsc.fetch_and_add(scnt.at[0], pcnt, subcore_id=0)

    gmax2 = (cnt + 2 * L - 1) >> 5

    def radix_pass(src, dst, shift):
        def outer(b, off):
            def body(g, off):
                off0 = off
                for h in range(2):
                    base = g * 2 * L + h * L
                    e16 = base + _iota()
                    valid = e16 < cnt
                    p16 = src[pl.ds(base, L)]
                    v16 = plsc.load_gather(
                        idx_all, [jnp.clip(p16, 0, NBUF - 1)]
                    )
                    key = ((v16 >> 7) - slab0) >> shift
                    m = ((key & 15) == b) & valid
                    mi = jnp.where(m, jnp.int32(1), jnp.int32(0))
                    incl = plsc.cumsum(mi)
                    tgt = jnp.where(m, off0 + incl - mi, jnp.int32(TRASH))
                    plsc.store_scatter(dst, [tgt], p16)
                    off0 = off0 + incl[L - 1]
                return off0
            return lax.fori_loop(0, gmax2, body, off)
        lax.fori_loop(0, 16, outer, jnp.int32(0))

    radix_pass(mA, mB, 0)
    radix_pass(mB, mA, 4)

    def runs_body(g, r):
        e16 = g * L + _iota()
        valid = e16 < cnt
        p16 = mA[pl.ds(g * L, L)]
        v16 = plsc.load_gather(idx_all, [jnp.clip(p16, 0, NBUF - 1)])
        pp = plsc.load_gather(mA, [jnp.clip(e16 - 1, 0, NBUF - 1)])
        vp = plsc.load_gather(idx_all, [jnp.clip(pp, 0, NBUF - 1)])
        ch = (((v16 >> 7) != (vp >> 7)) | (e16 == 0)) & valid
        mi = jnp.where(ch, jnp.int32(1), jnp.int32(0))
        incl = plsc.cumsum(mi)
        tgt = jnp.where(ch, r + incl - mi, jnp.int32(TRASH))
        plsc.store_scatter(mB, [tgt], e16)
        return r + incl[L - 1]

    nrun = lax.fori_loop(0, gmax, runs_body, jnp.int32(0))
    sent_tgt = jnp.where(_iota() == 0, nrun, jnp.int32(TRASH))
    plsc.store_scatter(mB, [sent_tgt], jnp.zeros((L,), jnp.int32) + cnt)

    def run_start(r):
        return mB[pl.ds(jnp.clip(r, 0, NBUF - L), L)][0]

    def elem_vocab(e):
        p = mA[pl.ds(jnp.clip(e, 0, NBUF - L), L)][0]
        v = idx_all[pl.ds(jnp.clip(p, 0, NBUF - L), L)][0]
        return p, v

    def fire_slab(r):
        _, v = elem_vocab(run_start(r))
        sb = v >> 7
        pltpu.async_copy(
            table_t.at[:, pl.ds(pl.multiple_of(sb * 128, 128), 128)],
            slabs.at[lax.rem(r, 8)],
            sems.at[lax.rem(r, 8)],
        )

    def drain_slab(r):
        pltpu.make_async_copy(
            table_t.at[:, pl.ds(0, 128)],
            slabs.at[lax.rem(r, 8)],
            sems.at[lax.rem(r, 8)],
        ).wait()

    for rr in range(6):
        @pl.when(nrun > rr)
        def _(rr=rr):
            fire_slab(rr)

    def flush_wait():
        pltpu.make_async_copy(
            rows_out.at[c, pl.ds(0, L), :], ext.at[0], sems.at[8]
        ).wait()

    def per_run(r, carry):
        w, fc = carry
        drain_slab(r)

        @pl.when(r + 6 < nrun)
        def _():
            fire_slab(r + 6)

        e0 = run_start(r)
        e1 = run_start(r + 1)
        ring = lax.rem(r, 8)

        def per_elem(e, carry):
            w, fc = carry
            p, v = elem_vocab(e)
            col = v & 127
            buf = lax.rem(w >> 4, 2)
            slot = w & 15

            @pl.when((slot == 0) & (fc >= 2))
            def _():
                flush_wait()

            for k in range(4):
                vals = plsc.load_gather(
                    slabs,
                    [ring + 0 * _iota(), k * L + _iota(), col + 0 * _iota()],
                )
                ext[buf, slot, pl.ds(k * L, L)] = vals
            pf = plsc.bitcast(jnp.zeros((L,), jnp.int32) + p, jnp.float32)
            rtgt = jnp.zeros((L,), jnp.int32) + slot
            ctgt = jnp.where(_iota() == 0, jnp.int32(DIM), jnp.int32(W - 1))
            plsc.store_scatter(ext.at[buf], [rtgt, ctgt], pf)

            @pl.when(w == 0)
            def _():
                first = [ext[0, 0, pl.ds(k * L, L)] for k in range(4)]
                for bb in range(2):
                    for sl in range(L):
                        for k in range(4):
                            ext[bb, sl, pl.ds(k * L, L)] = first[k]
                        rt = jnp.zeros((L,), jnp.int32) + sl
                        plsc.store_scatter(ext.at[bb], [rt, ctgt], pf)

            @pl.when(slot == 15)
            def _():
                dst = pl.multiple_of(start + (w & ~15), L)
                pltpu.async_copy(
                    ext.at[buf],
                    rows_out.at[c, pl.ds(dst, L), :],
                    sems.at[8],
                )

            fc = jnp.where(slot == 15, fc + 1, fc)
            return (w + 1, fc)

        return lax.fori_loop(e0, e1, per_elem, (w, fc))

    w, fc = lax.fori_loop(0, nrun, per_run, (jnp.int32(0), jnp.int32(0)))

    @pl.when((w & 15) != 0)
    def _():
        dst = pl.multiple_of(start + (w & ~15), L)
        pltpu.async_copy(
            ext.at[lax.rem(w >> 4, 2)],
            rows_out.at[c, pl.ds(dst, L), :],
            sems.at[8],
        )

    nflush = fc + jnp.where((w & 15) != 0, jnp.int32(1), jnp.int32(0))
    ndrain = jnp.minimum(nflush, jnp.int32(2))

    def drain_body(i, z):
        flush_wait()
        return z

    lax.fori_loop(0, ndrain, drain_body, jnp.int32(0))

    mv = (jnp.where(_iota() == 0, start, 0)
          + jnp.where(_iota() == 1, pcnt, 0))
    meta_v[0, pl.ds(0, L)] = mv
    pltpu.sync_copy(meta_v, meta_out.at[c, s])


@functools.partial(
    pl.kernel,
    mesh=_mesh,
    out_type=jax.ShapeDtypeStruct((BATCH, 128), jnp.float32),
    compiler_params=pltpu.CompilerParams(
        use_tc_tiling_on_sc=True, needs_layout_passes=False
    ),
    scratch_types=[
        pltpu.VMEM((8, 128), jnp.int32),
        pltpu.VMEM((4, L, W), jnp.float32),
        pltpu.VMEM((4, L, 128), jnp.float32),
        pltpu.VMEM((4, L), jnp.int32),
        pltpu.SemaphoreType.DMA((8,)),
    ],
)
def _k2(rows_in, meta_in, out_hbm, meta_v, buf, cbuf, pos_v, sems):
    c = lax.axis_index("c")
    s = lax.axis_index("s")
    pltpu.sync_copy(meta_in.at[c, s], meta_v)
    head = meta_v[0, pl.ds(0, L)]
    start = head[0]
    pcnt = head[1]
    nq = pcnt >> 4

    def fire(q):
        off = pl.multiple_of(start + q * L, L)
        pltpu.async_copy(
            rows_in.at[c, pl.ds(off, L), :], buf.at[lax.rem(q, 4)],
            sems.at[lax.rem(q, 4)],
        )

    def drain_read(q):
        pltpu.make_async_copy(
            rows_in.at[c, pl.ds(0, L), :], buf.at[lax.rem(q, 4)],
            sems.at[lax.rem(q, 4)],
        ).wait()

    def drain_scatter(q):
        pltpu.make_async_copy(
            cbuf.at[lax.rem(q, 4)],
            out_hbm.at[pos_v.at[lax.rem(q, 4)]],
            sems.at[4 + lax.rem(q, 4)],
        ).wait()

    @pl.when(nq > 0)
    def _():
        fire(0)

    @pl.when(nq > 1)
    def _():
        fire(1)

    def per_chunk(q, z):
        drain_read(q)
        bq = lax.rem(q, 4)
        pf = plsc.load_gather(
            buf, [bq + 0 * _iota(), _iota(), jnp.int32(DIM) + 0 * _iota()]
        )
        pos_v[bq, pl.ds(0, L)] = plsc.bitcast(pf, jnp.int32)
        for l in range(L):
            for k in range(4):
                cbuf[bq, l, pl.ds(k * L, L)] = buf[bq, l, pl.ds(k * L, L)]
        pltpu.async_copy(
            cbuf.at[bq],
            out_hbm.at[pos_v.at[bq]],
            sems.at[4 + bq],
        )

        @pl.when(q + 2 < nq)
        def _():
            @pl.when(q >= 2)
            def _():
                drain_scatter(q - 2)
            fire(q + 2)

        return z

    lax.fori_loop(0, nq, per_chunk, jnp.int32(0))

    def tail_drain(i, z):
        drain_scatter(i)
        return z

    lax.fori_loop(jnp.maximum(nq - 4, 0), nq, tail_drain, jnp.int32(0))


def kernel(x, table):
    idx = x.reshape(BATCH).astype(jnp.int32)
    rows, meta = _k1(table.T, idx)
    return _k2(rows, meta)[:, :DIM]
